# Initial kernel scaffold; baseline (speedup 1.0000x reference)
import jax
import jax.numpy as jnp
from jax.experimental import pallas as pl


def kernel(data, edge_index, emb, f_W, f_att_i, f_att_j, f_bias, f_bn_g, f_bn_b, d_W, d_att_i, d_att_j, d_bias, d_bn_g, d_bn_b, bn2_g, bn2_b, fl_W, fl_b, clf_W1, clf_b1, clf_W2, clf_b2, fus_W1, fus_b1, fus_W2, fus_b2, fus_W3, fus_b3):
    raise NotImplementedError("write your pallas kernel here")


# speedup vs baseline: 6.0287x; 6.0287x over previous
import functools

import jax
import jax.numpy as jnp
from jax.experimental import pallas as pl
from jax.experimental.pallas import tpu as pltpu

N = 10000
NP = 10112
D = 64
K = 20
EPS = 1e-5
ROWS = 128


def _topk_kernel(emb_blk, embT, nrm_row, nrm_col, out_ref):
    scores = jax.lax.dot_general(
        emb_blk[...], embT[...], (((1,), (0,)), ((), ())),
        preferred_element_type=jnp.float32)
    scores = scores / (nrm_row[...] * nrm_col[...])
    col = jax.lax.broadcasted_iota(jnp.int32, (ROWS, NP), 1)
    neg = jnp.float32(-jnp.inf)
    scores = jnp.where(col < N, scores, neg)
    lane = jax.lax.broadcasted_iota(jnp.int32, (ROWS, 128), 1)
    acc = jnp.zeros((ROWS, 128), jnp.int32)
    for t in range(K):
        m = jnp.max(scores, axis=1, keepdims=True)
        isel = jnp.min(jnp.where(scores >= m, col, NP), axis=1, keepdims=True)
        acc = jnp.where(lane == t, isel, acc)
        scores = jnp.where(col == isel, neg, scores)
    out_ref[...] = acc


def _learned_topk(emb):
    nrm = jnp.linalg.norm(emb, axis=-1)
    embp = jnp.concatenate([emb, jnp.zeros((NP - N, D), emb.dtype)], axis=0)
    nrmp = jnp.concatenate([nrm, jnp.ones((NP - N,), nrm.dtype)], axis=0)
    grid = NP // ROWS
    out = pl.pallas_call(
        _topk_kernel,
        grid=(grid,),
        in_specs=[
            pl.BlockSpec((ROWS, D), lambda i: (i, 0)),
            pl.BlockSpec((D, NP), lambda i: (0, 0)),
            pl.BlockSpec((ROWS, 1), lambda i: (i, 0)),
            pl.BlockSpec((1, NP), lambda i: (0, 0)),
        ],
        out_specs=pl.BlockSpec((ROWS, 128), lambda i: (i, 0)),
        out_shape=jax.ShapeDtypeStruct((NP, 128), jnp.int32),
    )(embp, embp.T, nrmp[:, None], nrmp[None, :])
    return out[:N, :K]


def _gat_layer(h, emb, att_i, att_j, bias, bn_g, bn_b, topk_idx):
    p = h @ att_i[:D] + (emb @ att_i[D:])[None, :]
    q = h @ att_j[:D] + (emb @ att_j[D:])[None, :]
    qg = jnp.take(q, topk_idx, axis=1)
    alpha = jax.nn.leaky_relu(p[:, :, None] + qg, 0.2)
    amax = jnp.max(alpha, axis=-1, keepdims=True)
    ex = jnp.exp(alpha - amax)
    a = ex / jnp.sum(ex, axis=-1, keepdims=True)
    hg = jnp.take(h, topk_idx, axis=1)
    out = jnp.einsum('bnk,bnkd->bnd', a, hg)
    out = out + bias
    out = out / jnp.sqrt(1.0 + EPS) * bn_g + bn_b
    return jax.nn.relu(out)


def kernel(data, edge_index, emb, f_W, f_att_i, f_att_j, f_bias, f_bn_g, f_bn_b,
           d_W, d_att_i, d_att_j, d_bias, d_bn_g, d_bn_b,
           bn2_g, bn2_b, fl_W, fl_b, clf_W1, clf_b1, clf_W2, clf_b2,
           fus_W1, fus_b1, fus_W2, fus_b2, fus_W3, fus_b3):
    del edge_index
    topk_idx = _learned_topk(emb)
    f_h = data @ f_W
    d_h = data @ d_W
    f_out = _gat_layer(f_h, emb, f_att_i, f_att_j, f_bias, f_bn_g, f_bn_b, topk_idx)
    d_out = _gat_layer(d_h, emb, d_att_i, d_att_j, d_bias, d_bn_g, d_bn_b, topk_idx)
    f_pool = f_out.mean(axis=1)
    det_pool = d_out.mean(axis=1)
    comb = jnp.concatenate([f_pool, det_pool], axis=1)
    h1 = jax.nn.relu(comb @ fus_W1 + fus_b1)
    h2 = jax.nn.relu(h1 @ fus_W2 + fus_b2)
    return jax.nn.sigmoid(h2 @ fus_W3 + fus_b3)
